# baseline (device time: 110150 ns/iter reference)
import jax
import jax.numpy as jnp
from jax import lax
from jax.experimental import pallas as pl
from jax.experimental.pallas import tpu as pltpu

N_DEV = 4
B, S, H, Dh, Dr = 2, 256, 16, 64, 32
D = 1024
BS = B * S


def kernel(x, Wdkv, Wuk, Wuv, Wq, Wqr, Wkr, Wo):
    def body(x_ref, wdkv_ref, wuk_ref, wuv_ref, wq_ref, wqr_ref, wkr_ref,
             wo_ref, out_ref, comm_ref, o_ref, send_sems, recv_sems):
        my = lax.axis_index("i")
        left = (my + N_DEV - 1) % N_DEV
        right = (my + 1) % N_DEV

        barrier_sem = pltpu.get_barrier_semaphore()
        for nbr in (left, right):
            pl.semaphore_signal(barrier_sem, inc=1, device_id=(nbr,),
                                device_id_type=pl.DeviceIdType.MESH)
        pl.semaphore_wait(barrier_sem, 2)

        xb = x_ref[...].reshape(BS, D).astype(jnp.bfloat16)

        c = jnp.dot(xb, wdkv_ref[...].astype(jnp.bfloat16),
                    preferred_element_type=jnp.float32).astype(jnp.bfloat16)
        kp = jnp.dot(c, wuk_ref[...].astype(jnp.bfloat16),
                     preferred_element_type=jnp.float32)
        vp = jnp.dot(c, wuv_ref[...].astype(jnp.bfloat16),
                     preferred_element_type=jnp.float32)
        comm_ref[0, :, :D] = kp.astype(jnp.bfloat16)
        comm_ref[0, :, D:] = vp.astype(jnp.bfloat16)

        for h in range(N_DEV - 1):
            rdma = pltpu.make_async_remote_copy(
                src_ref=comm_ref.at[h],
                dst_ref=comm_ref.at[h + 1],
                send_sem=send_sems.at[h],
                recv_sem=recv_sems.at[h],
                device_id=(right,),
                device_id_type=pl.DeviceIdType.MESH,
            )
            rdma.start()
            rdma.wait()

        kv = (comm_ref[0].astype(jnp.float32) + comm_ref[1].astype(jnp.float32)
              + comm_ref[2].astype(jnp.float32) + comm_ref[3].astype(jnp.float32))
        k_full = kv[:, :D].astype(jnp.bfloat16)
        v_full = kv[:, D:].astype(jnp.bfloat16)

        q = jnp.dot(xb, wq_ref[...].astype(jnp.bfloat16),
                    preferred_element_type=jnp.float32).astype(jnp.bfloat16)
        qr = jnp.dot(xb, wqr_ref[...].astype(jnp.bfloat16),
                     preferred_element_type=jnp.float32).astype(jnp.bfloat16)
        kr = jnp.dot(xb, wkr_ref[...].astype(jnp.bfloat16),
                     preferred_element_type=jnp.float32).astype(jnp.bfloat16)

        scale = (Dh + Dr) ** -0.5
        for b in range(B):
            rows = slice(b * S, (b + 1) * S)
            kr_b = kr[rows]
            for hh in range(H):
                qh = q[rows, hh * Dh:(hh + 1) * Dh]
                qrh = qr[rows, hh * Dr:(hh + 1) * Dr]
                kh = k_full[rows, hh * Dh:(hh + 1) * Dh]
                vh = v_full[rows, hh * Dh:(hh + 1) * Dh]
                s1 = lax.dot_general(qh, kh, (((1,), (1,)), ((), ())),
                                     preferred_element_type=jnp.float32)
                s2 = lax.dot_general(qrh, kr_b, (((1,), (1,)), ((), ())),
                                     preferred_element_type=jnp.float32)
                sc = (s1 + s2) * scale
                m = jnp.max(sc, axis=-1, keepdims=True)
                e = jnp.exp(sc - m)
                p = (e / jnp.sum(e, axis=-1, keepdims=True)).astype(jnp.bfloat16)
                o_ref[rows, hh * Dh:(hh + 1) * Dh] = jnp.dot(
                    p, vh, preferred_element_type=jnp.float32
                ).astype(jnp.bfloat16)

        out = jnp.dot(o_ref[...], wo_ref[...].astype(jnp.bfloat16),
                      preferred_element_type=jnp.float32)
        out_ref[...] = out.reshape(B, S, D)

    return pl.pallas_call(
        body,
        out_shape=jax.ShapeDtypeStruct((B, S, D), jnp.float32),
        in_specs=[pl.BlockSpec(memory_space=pltpu.VMEM)] * 8,
        out_specs=pl.BlockSpec(memory_space=pltpu.VMEM),
        scratch_shapes=[
            pltpu.VMEM((N_DEV, BS, 2 * D), jnp.bfloat16),
            pltpu.VMEM((BS, D), jnp.bfloat16),
            pltpu.SemaphoreType.DMA((N_DEV - 1,)),
            pltpu.SemaphoreType.DMA((N_DEV - 1,)),
        ],
        compiler_params=pltpu.CompilerParams(collective_id=0),
    )(x, Wdkv, Wuk, Wuv, Wq, Wqr, Wkr, Wo)


# device time: 56825 ns/iter; 1.9384x vs baseline; 1.9384x over previous
import jax
import jax.numpy as jnp
from jax import lax
from jax.experimental import pallas as pl
from jax.experimental.pallas import tpu as pltpu

N_DEV = 4
B, S, H, Dh, Dr = 2, 256, 16, 64, 32
D = 1024
BS = B * S
HB = D // N_DEV
HL = H // N_DEV
RB = Dr * HL


def kernel(x, Wdkv, Wuk, Wuv, Wq, Wqr, Wkr, Wo):
    def body(x_ref, wdkv_ref, wuk_ref, wuv_ref, wq_ref, wqr_ref, wkr_ref,
             wo_ref, out_ref, kv_ref, rs_send, rs_recv, ag_buf,
             rs_ssem, rs_rsem, ag_ssem, ag_rsem):
        my = lax.axis_index("i")
        left = (my + N_DEV - 1) % N_DEV
        right = (my + 1) % N_DEV

        barrier_sem = pltpu.get_barrier_semaphore()
        for nbr in (left, right):
            pl.semaphore_signal(barrier_sem, inc=1, device_id=(nbr,),
                                device_id_type=pl.DeviceIdType.MESH)
        pl.semaphore_wait(barrier_sem, 2)

        xb = x_ref[...].reshape(BS, D).astype(jnp.bfloat16)

        c = jnp.dot(xb, wdkv_ref[...].astype(jnp.bfloat16),
                    preferred_element_type=jnp.float32).astype(jnp.bfloat16)
        kv_ref[:, :D] = jnp.dot(c, wuk_ref[...].astype(jnp.bfloat16),
                                preferred_element_type=jnp.float32
                                ).astype(jnp.bfloat16)
        kv_ref[:, D:] = jnp.dot(c, wuv_ref[...].astype(jnp.bfloat16),
                                preferred_element_type=jnp.float32
                                ).astype(jnp.bfloat16)

        def chunk(j):
            kpart = kv_ref[:, pl.ds(j * HB, HB)]
            vpart = kv_ref[:, pl.ds(D + j * HB, HB)]
            return kpart, vpart

        def rs_rdma(t):
            return pltpu.make_async_remote_copy(
                src_ref=rs_send.at[t], dst_ref=rs_recv.at[t],
                send_sem=rs_ssem.at[t], recv_sem=rs_rsem.at[t],
                device_id=(right,), device_id_type=pl.DeviceIdType.MESH,
            )

        k0, v0 = chunk((my + N_DEV - 1) % N_DEV)
        rs_send[0, :, :HB] = k0
        rs_send[0, :, HB:] = v0
        rdma0 = rs_rdma(0)
        rdma0.start()

        q = jnp.dot(xb, wq_ref[:, pl.ds(my * HB, HB)].astype(jnp.bfloat16),
                    preferred_element_type=jnp.float32).astype(jnp.bfloat16)
        qr = jnp.dot(xb, wqr_ref[:, pl.ds(my * RB, RB)].astype(jnp.bfloat16),
                     preferred_element_type=jnp.float32).astype(jnp.bfloat16)
        kr = jnp.dot(xb, wkr_ref[...].astype(jnp.bfloat16),
                     preferred_element_type=jnp.float32).astype(jnp.bfloat16)
        rdma0.wait()

        k1, v1 = chunk((my + N_DEV - 2) % N_DEV)
        rs_send[1, :, :HB] = rs_recv[0, :, :HB] + k1
        rs_send[1, :, HB:] = rs_recv[0, :, HB:] + v1
        rdma1 = rs_rdma(1)
        rdma1.start()

        s2 = []
        for b in range(B):
            rows = slice(b * S, (b + 1) * S)
            kr_b = kr[rows]
            s2.append([
                lax.dot_general(qr[rows, hh * Dr:(hh + 1) * Dr], kr_b,
                                (((1,), (1,)), ((), ())),
                                preferred_element_type=jnp.float32)
                for hh in range(HL)
            ])
        rdma1.wait()

        k2, v2 = chunk((my + 1) % N_DEV)
        rs_send[2, :, :HB] = rs_recv[1, :, :HB] + k2
        rs_send[2, :, HB:] = rs_recv[1, :, HB:] + v2
        rdma2 = rs_rdma(2)
        rdma2.start()
        rdma2.wait()

        km, vm = chunk(my)
        k_my = rs_recv[2, :, :HB] + km
        v_my = rs_recv[2, :, HB:] + vm

        scale = (Dh + Dr) ** -0.5
        for b in range(B):
            rows = slice(b * S, (b + 1) * S)
            for hh in range(HL):
                cols = slice(hh * Dh, (hh + 1) * Dh)
                s1 = lax.dot_general(q[rows, cols], k_my[rows, cols],
                                     (((1,), (1,)), ((), ())),
                                     preferred_element_type=jnp.float32)
                sc = (s1 + s2[b][hh]) * scale
                m = jnp.max(sc, axis=-1, keepdims=True)
                e = jnp.exp(sc - m)
                p = (e / jnp.sum(e, axis=-1, keepdims=True)).astype(jnp.bfloat16)
                ag_buf[0, rows, cols] = jnp.dot(
                    p, v_my[rows, cols], preferred_element_type=jnp.float32
                ).astype(jnp.bfloat16)

        acc = None
        for h in range(N_DEV - 1):
            rdma = pltpu.make_async_remote_copy(
                src_ref=ag_buf.at[h], dst_ref=ag_buf.at[h + 1],
                send_sem=ag_ssem.at[h], recv_sem=ag_rsem.at[h],
                device_id=(right,), device_id_type=pl.DeviceIdType.MESH,
            )
            rdma.start()
            origin = (my + N_DEV - h) % N_DEV
            part = jnp.dot(
                ag_buf[h],
                wo_ref[pl.ds(origin * HB, HB), :].astype(jnp.bfloat16),
                preferred_element_type=jnp.float32)
            acc = part if acc is None else acc + part
            rdma.wait()
        origin = (my + 1) % N_DEV
        acc = acc + jnp.dot(
            ag_buf[N_DEV - 1],
            wo_ref[pl.ds(origin * HB, HB), :].astype(jnp.bfloat16),
            preferred_element_type=jnp.float32)
        out_ref[...] = acc.reshape(B, S, D)

    return pl.pallas_call(
        body,
        out_shape=jax.ShapeDtypeStruct((B, S, D), jnp.float32),
        in_specs=[pl.BlockSpec(memory_space=pltpu.VMEM)] * 8,
        out_specs=pl.BlockSpec(memory_space=pltpu.VMEM),
        scratch_shapes=[
            pltpu.VMEM((BS, 2 * D), jnp.bfloat16),
            pltpu.VMEM((N_DEV - 1, BS, 2 * HB), jnp.bfloat16),
            pltpu.VMEM((N_DEV - 1, BS, 2 * HB), jnp.bfloat16),
            pltpu.VMEM((N_DEV, BS, HB), jnp.bfloat16),
            pltpu.SemaphoreType.DMA((N_DEV - 1,)),
            pltpu.SemaphoreType.DMA((N_DEV - 1,)),
            pltpu.SemaphoreType.DMA((N_DEV - 1,)),
            pltpu.SemaphoreType.DMA((N_DEV - 1,)),
        ],
        compiler_params=pltpu.CompilerParams(collective_id=0),
    )(x, Wdkv, Wuk, Wuv, Wq, Wqr, Wkr, Wo)


# device time: 44845 ns/iter; 2.4562x vs baseline; 1.2671x over previous
import jax
import jax.numpy as jnp
from jax import lax
from jax.experimental import pallas as pl
from jax.experimental.pallas import tpu as pltpu

N_DEV = 4
B, S, H, Dh, Dr = 2, 256, 16, 64, 32
D = 1024
BS = B * S
HB = D // N_DEV
HL = H // N_DEV
RB = Dr * HL
HH = HB // 2


def kernel(x, Wdkv, Wuk, Wuv, Wq, Wqr, Wkr, Wo):
    def body(x_ref, wdkv_ref, wuk_ref, wuv_ref, wq_ref, wqr_ref, wkr_ref,
             wo_ref, out_ref, kv_ref, ksend, krecv, vsend, vrecv,
             agr, agl, sems):
        my = lax.axis_index("i")
        left = (my + N_DEV - 1) % N_DEV
        right = (my + 1) % N_DEV

        barrier_sem = pltpu.get_barrier_semaphore()
        for nbr in (left, right):
            pl.semaphore_signal(barrier_sem, inc=1, device_id=(nbr,),
                                device_id_type=pl.DeviceIdType.MESH)
        pl.semaphore_wait(barrier_sem, 2)

        xb = x_ref[...].reshape(BS, D).astype(jnp.bfloat16)

        c = jnp.dot(xb, wdkv_ref[...].astype(jnp.bfloat16),
                    preferred_element_type=jnp.float32).astype(jnp.bfloat16)
        kv_ref[:, :D] = jnp.dot(c, wuk_ref[...].astype(jnp.bfloat16),
                                preferred_element_type=jnp.float32
                                ).astype(jnp.bfloat16)
        kv_ref[:, D:] = jnp.dot(c, wuv_ref[...].astype(jnp.bfloat16),
                                preferred_element_type=jnp.float32
                                ).astype(jnp.bfloat16)

        def kchunk(j):
            return kv_ref[:, pl.ds(j * HB, HB)]

        def vchunk(j):
            return kv_ref[:, pl.ds(D + j * HB, HB)]

        def rs_pair(t):
            kr_ = pltpu.make_async_remote_copy(
                src_ref=ksend.at[t], dst_ref=krecv.at[t],
                send_sem=sems.at[0, t], recv_sem=sems.at[1, t],
                device_id=(right,), device_id_type=pl.DeviceIdType.MESH)
            vl_ = pltpu.make_async_remote_copy(
                src_ref=vsend.at[t], dst_ref=vrecv.at[t],
                send_sem=sems.at[2, t], recv_sem=sems.at[3, t],
                device_id=(left,), device_id_type=pl.DeviceIdType.MESH)
            return kr_, vl_

        ksend[0] = kchunk((my + N_DEV - 1) % N_DEV)
        vsend[0] = vchunk((my + 1) % N_DEV)
        k0, v0 = rs_pair(0)
        k0.start()
        v0.start()

        q = jnp.dot(xb, wq_ref[:, pl.ds(my * HB, HB)].astype(jnp.bfloat16),
                    preferred_element_type=jnp.float32).astype(jnp.bfloat16)
        qr = jnp.dot(xb, wqr_ref[:, pl.ds(my * RB, RB)].astype(jnp.bfloat16),
                     preferred_element_type=jnp.float32).astype(jnp.bfloat16)
        kr = jnp.dot(xb, wkr_ref[...].astype(jnp.bfloat16),
                     preferred_element_type=jnp.float32).astype(jnp.bfloat16)
        k0.wait()
        v0.wait()

        ksend[1] = krecv[0] + kchunk((my + N_DEV - 2) % N_DEV)
        vsend[1] = vrecv[0] + vchunk((my + 2) % N_DEV)
        k1, v1 = rs_pair(1)
        k1.start()
        v1.start()

        s2 = []
        for b in range(B):
            rows = slice(b * S, (b + 1) * S)
            kr_b = kr[rows]
            s2.append([
                lax.dot_general(qr[rows, hh * Dr:(hh + 1) * Dr], kr_b,
                                (((1,), (1,)), ((), ())),
                                preferred_element_type=jnp.float32)
                for hh in range(HL)
            ])
        k1.wait()
        v1.wait()

        ksend[2] = krecv[1] + kchunk((my + 1) % N_DEV)
        vsend[2] = vrecv[1] + vchunk((my + N_DEV - 1) % N_DEV)
        k2, v2 = rs_pair(2)
        k2.start()
        v2.start()
        k2.wait()
        v2.wait()

        k_my = krecv[2] + kchunk(my)
        v_my = vrecv[2] + vchunk(my)

        scale = (Dh + Dr) ** -0.5
        for b in range(B):
            rows = slice(b * S, (b + 1) * S)
            for hh in range(HL):
                cols = slice(hh * Dh, (hh + 1) * Dh)
                s1 = lax.dot_general(q[rows, cols], k_my[rows, cols],
                                     (((1,), (1,)), ((), ())),
                                     preferred_element_type=jnp.float32)
                sc = (s1 + s2[b][hh]) * scale
                m = jnp.max(sc, axis=-1, keepdims=True)
                e = jnp.exp(sc - m)
                p = (e / jnp.sum(e, axis=-1, keepdims=True)).astype(jnp.bfloat16)
                o_bh = jnp.dot(p, v_my[rows, cols],
                               preferred_element_type=jnp.float32
                               ).astype(jnp.bfloat16)
                if hh < HL // 2:
                    agr[0, rows, cols] = o_bh
                else:
                    agl[0, rows, slice(cols.start - HH, cols.stop - HH)] = o_bh

        wo = wo_ref
        acc = None

        def wo_part(buf, origin, half):
            return jnp.dot(
                buf,
                wo[pl.ds(origin * HB + half * HH, HH), :].astype(jnp.bfloat16),
                preferred_element_type=jnp.float32)

        for h in range(N_DEV - 1):
            ra = pltpu.make_async_remote_copy(
                src_ref=agr.at[h], dst_ref=agr.at[h + 1],
                send_sem=sems.at[4, h], recv_sem=sems.at[5, h],
                device_id=(right,), device_id_type=pl.DeviceIdType.MESH)
            la = pltpu.make_async_remote_copy(
                src_ref=agl.at[h], dst_ref=agl.at[h + 1],
                send_sem=sems.at[6, h], recv_sem=sems.at[7, h],
                device_id=(left,), device_id_type=pl.DeviceIdType.MESH)
            ra.start()
            la.start()
            part = (wo_part(agr[h], (my + N_DEV - h) % N_DEV, 0)
                    + wo_part(agl[h], (my + h) % N_DEV, 1))
            acc = part if acc is None else acc + part
            ra.wait()
            la.wait()
        acc = acc + wo_part(agr[N_DEV - 1], (my + 1) % N_DEV, 0)
        acc = acc + wo_part(agl[N_DEV - 1], (my + N_DEV - 1) % N_DEV, 1)
        out_ref[...] = acc.reshape(B, S, D)

    return pl.pallas_call(
        body,
        out_shape=jax.ShapeDtypeStruct((B, S, D), jnp.float32),
        in_specs=[pl.BlockSpec(memory_space=pltpu.VMEM)] * 8,
        out_specs=pl.BlockSpec(memory_space=pltpu.VMEM),
        scratch_shapes=[
            pltpu.VMEM((BS, 2 * D), jnp.bfloat16),
            pltpu.VMEM((N_DEV - 1, BS, HB), jnp.bfloat16),
            pltpu.VMEM((N_DEV - 1, BS, HB), jnp.bfloat16),
            pltpu.VMEM((N_DEV - 1, BS, HB), jnp.bfloat16),
            pltpu.VMEM((N_DEV - 1, BS, HB), jnp.bfloat16),
            pltpu.VMEM((N_DEV, BS, HH), jnp.bfloat16),
            pltpu.VMEM((N_DEV, BS, HH), jnp.bfloat16),
            pltpu.SemaphoreType.DMA((8, N_DEV - 1)),
        ],
        compiler_params=pltpu.CompilerParams(collective_id=0),
    )(x, Wdkv, Wuk, Wuv, Wq, Wqr, Wkr, Wo)
